# baseline (speedup 1.0000x reference)
import jax
import jax.numpy as jnp
import numpy as np
from jax import lax
from jax.experimental import pallas as pl
from jax.experimental.pallas import tpu as pltpu
from jax.experimental.pallas import tpu_sc as plsc

NUM_CLASSES = 1000
SMOOTHING = 0.1
FILL = float(np.float32(SMOOTHING / NUM_CLASSES))
PEAK = float(np.float32(np.float32(SMOOTHING / NUM_CLASSES) + np.float32(1.0 - SMOOTHING)))

NC, NS, L = 2, 16, 16
NW = NC * NS
BATCH = 16384
RPW = BATCH // NW
R = 32
NCHUNK = RPW // R


def _sc_body(tgt_hbm, fill_hbm, out_hbm, tgt_v, buf0, buf1, sem0, sem1, semf):
    wid = lax.axis_index("s") * NC + lax.axis_index("c")
    base_row = wid * RPW

    fa = pltpu.async_copy(fill_hbm, buf0, semf)
    fb = pltpu.async_copy(fill_hbm, buf1, semf)
    pltpu.sync_copy(tgt_hbm.at[pl.ds(base_row, RPW)], tgt_v)
    fa.wait()
    fb.wait()

    fill_vec = jnp.full((L,), FILL, jnp.float32)
    peak_vec = jnp.full((L,), PEAK, jnp.float32)
    lanes = lax.iota(jnp.int32, L)

    bufs = (buf0, buf1)
    sems = (sem0, sem1)

    def indices(b):
        out = []
        for k in range(R // L):
            t = tgt_v[pl.ds(b * R + k * L, L)]
            out.append((k * L + lanes, t))
        return out

    copies = [None, None]
    for b in range(NCHUNK):
        p = b % 2
        if copies[p] is not None:
            copies[p].wait()
            for rows, cols in indices(b - 2):
                plsc.store_scatter(bufs[p], [rows, cols], fill_vec)
        for rows, cols in indices(b):
            plsc.store_scatter(bufs[p], [rows, cols], peak_vec)
        dst = out_hbm.at[pl.ds(base_row + b * R, R), :]
        copies[p] = pltpu.async_copy(bufs[p], dst, sems[p])
    copies[0].wait()
    copies[1].wait()


def kernel(target, pred):
    batch = target.shape[0]
    fill_const = jnp.full((R, NUM_CLASSES), FILL, jnp.float32)
    mesh = plsc.VectorSubcoreMesh(core_axis_name="c", subcore_axis_name="s")
    return pl.kernel(
        _sc_body,
        out_type=jax.ShapeDtypeStruct((batch, NUM_CLASSES), jnp.float32),
        mesh=mesh,
        compiler_params=pltpu.CompilerParams(
            needs_layout_passes=False, use_tc_tiling_on_sc=True),
        scratch_types=[
            pltpu.VMEM((RPW,), jnp.int32),
            pltpu.VMEM((R, NUM_CLASSES), jnp.float32),
            pltpu.VMEM((R, NUM_CLASSES), jnp.float32),
            pltpu.SemaphoreType.DMA,
            pltpu.SemaphoreType.DMA,
            pltpu.SemaphoreType.DMA,
        ],
    )(target, fill_const)


# speedup vs baseline: 1.7794x; 1.7794x over previous
import jax
import jax.numpy as jnp
import numpy as np
from jax import lax
from jax.experimental import pallas as pl
from jax.experimental.pallas import tpu as pltpu
from jax.experimental.pallas import tpu_sc as plsc

NUM_CLASSES = 1000
SMOOTHING = 0.1
FILL = float(np.float32(SMOOTHING / NUM_CLASSES))
PEAK = float(np.float32(np.float32(SMOOTHING / NUM_CLASSES) + np.float32(1.0 - SMOOTHING)))

NC, NS, L = 2, 16, 16
NW = NC * NS
BATCH = 16384
BPW = BATCH // NW
CH = 96
CHUNKS = [(lo, min(CH, NUM_CLASSES - lo)) for lo in range(0, NUM_CLASSES, CH)]


def _sc_body(tgt_hbm, fill_hbm, out_hbm, tgt_v, buf0, buf1, sem0, sem1, semf):
    wid = lax.axis_index("s") * NC + lax.axis_index("c")
    i0 = wid * BPW

    fa = pltpu.async_copy(fill_hbm, buf0, semf)
    fb = pltpu.async_copy(fill_hbm, buf1, semf)
    pltpu.sync_copy(tgt_hbm.at[pl.ds(i0, BPW)], tgt_v)
    fa.wait()
    fb.wait()

    fill_vec = jnp.full((L,), FILL, jnp.float32)
    peak_vec = jnp.full((L,), PEAK, jnp.float32)
    lanes = lax.iota(jnp.int32, L)

    bufs = (buf0, buf1)
    sems = (sem0, sem1)

    def apply(buf, lo, h, vec):
        for k in range(BPW // L):
            t = tgt_v[pl.ds(k * L, L)]
            m = (t >= lo) & (t < lo + h)
            row = jnp.where(m, t - lo, 0)
            plsc.store_scatter(buf, [row, k * L + lanes], vec, mask=m)

    copies = [None, None]
    prev = [None, None]
    for ci, (lo, h) in enumerate(CHUNKS):
        p = ci % 2
        if copies[p] is not None:
            copies[p].wait()
            plo, ph = prev[p]
            apply(bufs[p], plo, ph, fill_vec)
        apply(bufs[p], lo, h, peak_vec)
        dst = out_hbm.at[pl.ds(lo, h), pl.ds(i0, BPW)]
        src = bufs[p] if h == CH else bufs[p].at[pl.ds(0, h), :]
        copies[p] = pltpu.async_copy(src, dst, sems[p])
        prev[p] = (lo, h)
    copies[0].wait()
    copies[1].wait()


def kernel(target, pred):
    batch = target.shape[0]
    fill_const = jnp.full((CH, BPW), FILL, jnp.float32)
    mesh = plsc.VectorSubcoreMesh(core_axis_name="c", subcore_axis_name="s")
    out_t = pl.kernel(
        _sc_body,
        out_type=jax.ShapeDtypeStruct((NUM_CLASSES, batch), jnp.float32),
        mesh=mesh,
        compiler_params=pltpu.CompilerParams(needs_layout_passes=False),
        scratch_types=[
            pltpu.VMEM((BPW,), jnp.int32),
            pltpu.VMEM((CH, BPW), jnp.float32),
            pltpu.VMEM((CH, BPW), jnp.float32),
            pltpu.SemaphoreType.DMA,
            pltpu.SemaphoreType.DMA,
            pltpu.SemaphoreType.DMA,
        ],
    )(target, fill_const)
    return out_t.T


# speedup vs baseline: 1.8602x; 1.0454x over previous
import jax
import jax.numpy as jnp
import numpy as np
from jax import lax
from jax.experimental import pallas as pl
from jax.experimental.pallas import tpu as pltpu
from jax.experimental.pallas import tpu_sc as plsc

NUM_CLASSES = 1000
SMOOTHING = 0.1
FILL = float(np.float32(SMOOTHING / NUM_CLASSES))
PEAK = float(np.float32(np.float32(SMOOTHING / NUM_CLASSES) + np.float32(1.0 - SMOOTHING)))

NC, NS, L = 2, 16, 16
NW = NC * NS
BATCH = 16384
BPW = BATCH // NW
CH = 120
CHUNKS = [(lo, min(CH, NUM_CLASSES - lo)) for lo in range(0, NUM_CLASSES, CH)]


def _sc_body(tgt_hbm, fill_hbm, out_hbm, tgt_v, buf0, buf1, sem0, sem1, semf):
    wid = lax.axis_index("s") * NC + lax.axis_index("c")
    i0 = wid * BPW

    fa = pltpu.async_copy(fill_hbm, buf0, semf)
    fb = pltpu.async_copy(fill_hbm, buf1, semf)
    pltpu.sync_copy(tgt_hbm.at[pl.ds(i0, BPW)], tgt_v)
    fa.wait()
    fb.wait()

    fill_vec = jnp.full((L,), FILL, jnp.float32)
    peak_vec = jnp.full((L,), PEAK, jnp.float32)
    lanes = lax.iota(jnp.int32, L)

    bufs = (buf0, buf1)
    sems = (sem0, sem1)

    def patch(buf, lo, h, prev_chunk):
        for k in range(BPW // L):
            t = tgt_v[pl.ds(k * L, L)]
            col = k * L + lanes
            if prev_chunk is not None:
                plo, ph = prev_chunk
                prow = t - plo
                pm = prow.astype(jnp.uint32) < jnp.uint32(ph)
                plsc.store_scatter(buf, [prow, col], fill_vec, mask=pm)
            row = t - lo
            m = row.astype(jnp.uint32) < jnp.uint32(h)
            plsc.store_scatter(buf, [row, col], peak_vec, mask=m)

    copies = [None, None]
    prev = [None, None]
    for ci, (lo, h) in enumerate(CHUNKS):
        p = ci % 2
        if copies[p] is not None:
            copies[p].wait()
        patch(bufs[p], lo, h, prev[p])
        dst = out_hbm.at[pl.ds(lo, h), pl.ds(i0, BPW)]
        src = bufs[p] if h == CH else bufs[p].at[pl.ds(0, h), :]
        copies[p] = pltpu.async_copy(src, dst, sems[p])
        prev[p] = (lo, h)
    copies[0].wait()
    copies[1].wait()


def kernel(target, pred):
    batch = target.shape[0]
    fill_const = jnp.full((CH, BPW), FILL, jnp.float32)
    mesh = plsc.VectorSubcoreMesh(core_axis_name="c", subcore_axis_name="s")
    out_t = pl.kernel(
        _sc_body,
        out_type=jax.ShapeDtypeStruct((NUM_CLASSES, batch), jnp.float32),
        mesh=mesh,
        compiler_params=pltpu.CompilerParams(needs_layout_passes=False),
        scratch_types=[
            pltpu.VMEM((BPW,), jnp.int32),
            pltpu.VMEM((CH, BPW), jnp.float32),
            pltpu.VMEM((CH, BPW), jnp.float32),
            pltpu.SemaphoreType.DMA,
            pltpu.SemaphoreType.DMA,
            pltpu.SemaphoreType.DMA,
        ],
    )(target, fill_const)
    return out_t.T


# speedup vs baseline: 1.9796x; 1.0642x over previous
import jax
import jax.numpy as jnp
import numpy as np
from jax import lax
from jax.experimental import pallas as pl
from jax.experimental.pallas import tpu as pltpu
from jax.experimental.pallas import tpu_sc as plsc

NUM_CLASSES = 1000
SMOOTHING = 0.1
FILL = float(np.float32(SMOOTHING / NUM_CLASSES))
PEAK = float(np.float32(np.float32(SMOOTHING / NUM_CLASSES) + np.float32(1.0 - SMOOTHING)))

NC, NS, L = 2, 16, 16
NW = NC * NS
BATCH = 16384
BPW = BATCH // NW
CH = 120
CHUNKS = [(lo, min(CH, NUM_CLASSES - lo)) for lo in range(0, NUM_CLASSES, CH)]


def _sc_body(tgt_hbm, fill_hbm, out_hbm, tgt_v, buf0, buf1, sem0, sem1, semf):
    wid = lax.axis_index("s") * NC + lax.axis_index("c")
    i0 = wid * BPW

    fa = pltpu.async_copy(fill_hbm, buf0, semf)
    fb = pltpu.async_copy(fill_hbm, buf1, semf)
    pltpu.sync_copy(tgt_hbm.at[pl.ds(i0, BPW)], tgt_v)
    fa.wait()
    fb.wait()

    fill_vec = jnp.full((L,), FILL, jnp.float32)
    peak_vec = jnp.full((L,), PEAK, jnp.float32)
    lanes = lax.iota(jnp.int32, L)

    bufs = (buf0, buf1)
    sems = (sem0, sem1)

    def patch(buf, lo, h, prev_chunk):
        def body(k, carry):
            t = tgt_v[pl.ds(k * L, L)]
            col = k * L + lanes
            if prev_chunk is not None:
                plo, ph = prev_chunk
                prow = t - plo
                pm = prow.astype(jnp.uint32) < jnp.uint32(ph)
                plsc.store_scatter(buf, [prow, col], fill_vec, mask=pm)
            row = t - lo
            m = row.astype(jnp.uint32) < jnp.uint32(h)
            plsc.store_scatter(buf, [row, col], peak_vec, mask=m)
            return carry
        lax.fori_loop(0, BPW // L, body, 0, unroll=4)

    copies = [None, None]
    prev = [None, None]
    for ci, (lo, h) in enumerate(CHUNKS):
        p = ci % 2
        if copies[p] is not None:
            copies[p].wait()
        patch(bufs[p], lo, h, prev[p])
        dst = out_hbm.at[pl.ds(lo, h), pl.ds(i0, BPW)]
        src = bufs[p] if h == CH else bufs[p].at[pl.ds(0, h), :]
        copies[p] = pltpu.async_copy(src, dst, sems[p])
        prev[p] = (lo, h)
    copies[0].wait()
    copies[1].wait()


def kernel(target, pred):
    batch = target.shape[0]
    fill_const = jnp.full((CH, BPW), FILL, jnp.float32)
    mesh = plsc.VectorSubcoreMesh(core_axis_name="c", subcore_axis_name="s")
    out_t = pl.kernel(
        _sc_body,
        out_type=jax.ShapeDtypeStruct((NUM_CLASSES, batch), jnp.float32),
        mesh=mesh,
        compiler_params=pltpu.CompilerParams(needs_layout_passes=False),
        scratch_types=[
            pltpu.VMEM((BPW,), jnp.int32),
            pltpu.VMEM((CH, BPW), jnp.float32),
            pltpu.VMEM((CH, BPW), jnp.float32),
            pltpu.SemaphoreType.DMA,
            pltpu.SemaphoreType.DMA,
            pltpu.SemaphoreType.DMA,
        ],
    )(target, fill_const)
    return out_t.T
